# Initial kernel scaffold; baseline (speedup 1.0000x reference)
import jax
import jax.numpy as jnp
from jax.experimental import pallas as pl


def kernel(x, e, g, edges, node_idx, edge_idx, W_e1, b_e1, W_e2, b_e2, W_n1, b_n1, ln_g, ln_b, W_n2, b_n2, W_g1, b_g1, W_g2, b_g2, W_g3, b_g3):
    raise NotImplementedError("write your pallas kernel here")


# speedup vs baseline: 9.2836x; 9.2836x over previous
import functools
import jax
import jax.numpy as jnp
from jax import lax
from jax.experimental import pallas as pl
from jax.experimental.pallas import tpu as pltpu
from jax.experimental.pallas import tpu_sc as plsc

N = 10000
E = 320000
G = 64
H = 10
NP = 10240
NW = 32
EPW = E // NW
BE = 3200
BN = 2048
F32 = jnp.float32
I32 = jnp.int32


def _a1_body(eT_ref, eidx_ref, gT_ref, wa1_ref, wg_ref, out_ref):
    gpT = jnp.dot(wg_ref[...], gT_ref[...], preferred_element_type=F32)
    iot = lax.broadcasted_iota(I32, (G, BE), 0).astype(F32)
    oh = (iot == eidx_ref[...]).astype(F32)
    out_ref[...] = (jnp.dot(wa1_ref[...], eT_ref[...], preferred_element_type=F32)
                    + jnp.dot(gpT, oh, preferred_element_type=F32))


def _a2_body(xT_ref, wa2_ref, out_ref):
    out_ref[...] = jnp.dot(wa2_ref[...], xT_ref[...], preferred_element_type=F32)


def _b_body(baseT, xallT, srch, dsth, eidxh, wth,
            eo, e2np, e2gp,
            src_v, dst_v, eidx_v, acc_v, base_v, xs_v, xd_v, e2n_v, e2g_v, wt_v):
    wid = lax.axis_index("s") * 2 + lax.axis_index("c")
    be = wid * EPW
    pltpu.sync_copy(srch.at[pl.ds(be, EPW)], src_v)
    pltpu.sync_copy(dsth.at[pl.ds(be, EPW)], dst_v)
    pltpu.sync_copy(eidxh.at[pl.ds(be, EPW)], eidx_v)
    pltpu.sync_copy(wth, wt_v)
    nch = EPW // 16

    b_vec = wt_v[H, :]

    def zacc(i, c):
        acc_v[pl.ds(i * 16, 16)] = b_vec
        return c
    lax.fori_loop(0, nch, zacc, 0)

    zero = jnp.zeros((16,), F32)

    def zn(i, c):
        e2n_v[pl.ds(i * 16, 16)] = zero
        return c
    lax.fori_loop(0, NP // 16, zn, 0)
    for j in range(G // 16):
        e2g_v[pl.ds(j * 16, 16)] = zero

    for d in range(H):
        pltpu.sync_copy(baseT.at[d, pl.ds(be, EPW)], base_v)
        pltpu.sync_copy(xallT.at[d], xs_v)
        pltpu.sync_copy(xallT.at[H + d], xd_v)
        w_vec = wt_v[d, :]

        def ebody(i, c):
            ds_ = pl.ds(i * 16, 16)
            hh = (base_v[ds_]
                  + plsc.load_gather(xs_v, [src_v[ds_]])
                  + plsc.load_gather(xd_v, [dst_v[ds_]]))
            acc_v[ds_] = acc_v[ds_] + jnp.maximum(hh, 0.0) * w_vec
            return c
        lax.fori_loop(0, nch, ebody, 0)

    pltpu.sync_copy(acc_v, eo.at[pl.ds(be, EPW)])

    def sbody(i, c):
        ds_ = pl.ds(i * 16, 16)
        v = acc_v[ds_]
        plsc.addupdate_scatter(e2n_v, [dst_v[ds_]], v)
        plsc.addupdate_scatter(e2g_v, [eidx_v[ds_]], v)
        return c
    lax.fori_loop(0, nch, sbody, 0)

    pltpu.sync_copy(e2n_v, e2np.at[wid])
    pltpu.sync_copy(e2g_v, e2gp.at[wid])


def _c_body(xT_ref, gT_ref, parts_ref, nrow_ref, ncol_ref,
            wnx_ref, wng_ref, wcol_ref, lng_ref, lnb_ref, wn2_ref,
            nout_ref, n2g_ref):
    i = pl.program_id(0)
    h = jnp.dot(wnx_ref[...], xT_ref[...], preferred_element_type=F32)
    gnT = jnp.dot(wng_ref[...], gT_ref[...], preferred_element_type=F32)
    ohg = (lax.broadcasted_iota(I32, (G, BN), 0).astype(F32) == nrow_ref[...]).astype(F32)
    h = h + jnp.dot(gnT, ohg, preferred_element_type=F32)
    e2n_row = jnp.sum(parts_ref[...], axis=0, keepdims=True)
    h = h + wcol_ref[...] * e2n_row
    mu = jnp.mean(h, axis=0, keepdims=True)
    d0 = h - mu
    var = jnp.mean(d0 * d0, axis=0, keepdims=True)
    y = d0 * lax.rsqrt(var + 1e-5) * lng_ref[...] + lnb_ref[...]
    r = jnp.maximum(y, 0.0)
    r1 = jnp.concatenate([r, jnp.ones((1, BN), F32)], axis=0)
    no = jnp.dot(wn2_ref[...], r1, preferred_element_type=F32)
    gid = lax.broadcasted_iota(I32, (1, BN), 1) + i * BN
    no = jnp.where(gid < N, no, 0.0)
    nout_ref[...] = no
    ohn = (lax.broadcasted_iota(I32, (BN, G), 1).astype(F32) == ncol_ref[...]).astype(F32)
    part = jnp.dot(no, ohn, preferred_element_type=F32)

    @pl.when(i == 0)
    def _():
        n2g_ref[...] = part

    @pl.when(i > 0)
    def _():
        n2g_ref[...] = n2g_ref[...] + part


def _d_body(gT_ref, n2g_ref, e2gp_ref, wg1_ref, wg2_ref, wg3_ref, out_ref):
    e2g_row = jnp.sum(e2gp_ref[...], axis=0, keepdims=True)
    ones = jnp.ones((1, G), F32)
    gin = jnp.concatenate([gT_ref[...], n2g_ref[...], e2g_row, ones], axis=0)
    h1 = jnp.maximum(jnp.dot(wg1_ref[...], gin, preferred_element_type=F32), 0.0)
    h1 = jnp.concatenate([h1, ones], axis=0)
    h2 = jnp.maximum(jnp.dot(wg2_ref[...], h1, preferred_element_type=F32), 0.0)
    h2 = jnp.concatenate([h2, ones], axis=0)
    out_ref[...] = jnp.dot(wg3_ref[...], h2, preferred_element_type=F32)


@jax.jit
def kernel(x, e, g, edges, node_idx, edge_idx, W_e1, b_e1, W_e2, b_e2,
           W_n1, b_n1, ln_g, ln_b, W_n2, b_n2,
           W_g1, b_g1, W_g2, b_g2, W_g3, b_g3):
    src = edges[0]
    dst = edges[1]

    eT24 = jnp.concatenate([e.T, jnp.ones((1, E), F32), jnp.zeros((7, E), F32)], axis=0)
    xT = x.T
    xT136 = jnp.concatenate(
        [xT, jnp.ones((1, N), F32), jnp.zeros((7, N), F32)], axis=0)
    xT136 = jnp.pad(xT136, ((0, 0), (0, NP - N)))
    gT = g.T

    eidx_row = edge_idx.astype(F32)[None, :]
    nidx_pad = jnp.pad(node_idx, (0, NP - N), constant_values=G - 1)
    nrow = nidx_pad.astype(F32)[None, :]
    ncol = nidx_pad.astype(F32)[:, None]

    WA1 = jnp.zeros((16, 24), F32)
    WA1 = WA1.at[:H, :16].set(W_e1[:16].T).at[:H, 16].set(b_e1)
    WG16 = jnp.zeros((16, 32), F32).at[:H].set(W_e1[272:304].T)
    WA2 = jnp.zeros((24, 136), F32)
    WA2 = WA2.at[:H, :128].set(W_e1[16:144].T).at[H:2 * H, :128].set(W_e1[144:272].T)
    WNX = jnp.zeros((H, 136), F32).at[:, :128].set(W_n1[:128].T).at[:, 128].set(b_n1)
    WNG = W_n1[128:160].T
    wcol = W_n1[160][:, None]
    lng = ln_g[:, None]
    lnb = ln_b[:, None]
    WN2 = jnp.concatenate([W_n2.T, b_n2[None, :]], axis=1)
    WG1 = jnp.concatenate([W_g1.T, b_g1[:, None]], axis=1)
    WG2 = jnp.concatenate([W_g2.T, b_g2[:, None]], axis=1)
    WG3 = jnp.concatenate([W_g3.T, b_g3[:, None]], axis=1)
    wtab_e = jnp.zeros((16,), F32).at[:H].set(W_e2[:, 0]).at[H].set(b_e2[0])
    wtab_e = jnp.tile(wtab_e[:, None], (1, 16))

    baseT = pl.pallas_call(
        _a1_body,
        grid=(E // BE,),
        in_specs=[
            pl.BlockSpec((24, BE), lambda i: (0, i)),
            pl.BlockSpec((1, BE), lambda i: (0, i)),
            pl.BlockSpec((32, G), lambda i: (0, 0)),
            pl.BlockSpec((16, 24), lambda i: (0, 0)),
            pl.BlockSpec((16, 32), lambda i: (0, 0)),
        ],
        out_specs=pl.BlockSpec((16, BE), lambda i: (0, i)),
        out_shape=jax.ShapeDtypeStruct((16, E), F32),
    )(eT24, eidx_row, gT, WA1, WG16)

    xallT = pl.pallas_call(
        _a2_body,
        grid=(NP // BN,),
        in_specs=[
            pl.BlockSpec((136, BN), lambda i: (0, i)),
            pl.BlockSpec((24, 136), lambda i: (0, 0)),
        ],
        out_specs=pl.BlockSpec((24, BN), lambda i: (0, i)),
        out_shape=jax.ShapeDtypeStruct((24, NP), F32),
    )(xT136, WA2)

    mesh = plsc.VectorSubcoreMesh(core_axis_name="c", subcore_axis_name="s")
    scb = functools.partial(
        pl.kernel,
        out_type=[
            jax.ShapeDtypeStruct((E,), F32),
            jax.ShapeDtypeStruct((NW, NP), F32),
            jax.ShapeDtypeStruct((NW, G), F32),
        ],
        mesh=mesh,
        compiler_params=pltpu.CompilerParams(
            use_tc_tiling_on_sc=False, needs_layout_passes=False),
        scratch_types=[
            pltpu.VMEM((EPW,), I32),
            pltpu.VMEM((EPW,), I32),
            pltpu.VMEM((EPW,), I32),
            pltpu.VMEM((EPW,), F32),
            pltpu.VMEM((EPW,), F32),
            pltpu.VMEM((NP,), F32),
            pltpu.VMEM((NP,), F32),
            pltpu.VMEM((NP,), F32),
            pltpu.VMEM((G,), F32),
            pltpu.VMEM((16, 16), F32),
        ],
    )
    edge_out_flat, e2n_parts, e2g_parts = scb(_b_body)(
        baseT, xallT, src, dst, edge_idx, wtab_e)

    node_outT, n2g = pl.pallas_call(
        _c_body,
        grid=(NP // BN,),
        in_specs=[
            pl.BlockSpec((136, BN), lambda i: (0, i)),
            pl.BlockSpec((32, G), lambda i: (0, 0)),
            pl.BlockSpec((NW, BN), lambda i: (0, i)),
            pl.BlockSpec((1, BN), lambda i: (0, i)),
            pl.BlockSpec((BN, 1), lambda i: (i, 0)),
            pl.BlockSpec((H, 136), lambda i: (0, 0)),
            pl.BlockSpec((H, 32), lambda i: (0, 0)),
            pl.BlockSpec((H, 1), lambda i: (0, 0)),
            pl.BlockSpec((H, 1), lambda i: (0, 0)),
            pl.BlockSpec((H, 1), lambda i: (0, 0)),
            pl.BlockSpec((1, H + 1), lambda i: (0, 0)),
        ],
        out_specs=[
            pl.BlockSpec((1, BN), lambda i: (0, i)),
            pl.BlockSpec((1, G), lambda i: (0, 0)),
        ],
        out_shape=[
            jax.ShapeDtypeStruct((1, NP), F32),
            jax.ShapeDtypeStruct((1, G), F32),
        ],
    )(xT136, gT, e2n_parts, nrow, ncol, WNX, WNG, wcol, lng, lnb, WN2)

    globT = pl.pallas_call(
        _d_body,
        grid=(1,),
        in_specs=[
            pl.BlockSpec((32, G), lambda i: (0, 0)),
            pl.BlockSpec((1, G), lambda i: (0, 0)),
            pl.BlockSpec((NW, G), lambda i: (0, 0)),
            pl.BlockSpec((H, 35), lambda i: (0, 0)),
            pl.BlockSpec((H, H + 1), lambda i: (0, 0)),
            pl.BlockSpec((1, H + 1), lambda i: (0, 0)),
        ],
        out_specs=pl.BlockSpec((1, G), lambda i: (0, 0)),
        out_shape=jax.ShapeDtypeStruct((1, G), F32),
    )(gT, n2g, e2g_parts, WG1, WG2, WG3)

    edge_out = edge_out_flat[:, None]
    node_out = node_outT.reshape(NP, 1)[:N]
    glob_out = globT.reshape(G, 1)
    return (edge_out, node_out, glob_out)
